# baseline (speedup 1.0000x reference)
import functools

import jax
import jax.numpy as jnp
from jax import lax
from jax.experimental import pallas as pl
from jax.experimental.pallas import tpu as pltpu
from jax.experimental.pallas import tpu_sc as plsc

N = 10000
NPAD = 10112
E = 320000
K = 128
NCHUNKP = 2560
EPAD = NCHUNKP * K - E
NCW = NCHUNKP // 32
NSUB = 16
TILE_ROWS = NPAD // NSUB
NROWSTEP = TILE_ROWS // K
TAIL_ROWS = TILE_ROWS - NROWSTEP * K
BM = 1264


def _staged_copy(src_ref, dst_ref, bounce, base):
    def step(j, carry):
        off = pl.multiple_of(base + j * K, 8)
        pltpu.sync_copy(src_ref.at[pl.ds(off, K)], bounce)
        pltpu.sync_copy(bounce, dst_ref.at[pl.ds(off, K)])
        return carry
    lax.fori_loop(0, NROWSTEP, step, 0)
    toff = pl.multiple_of(base + NROWSTEP * K, 8)
    pltpu.sync_copy(src_ref.at[pl.ds(toff, TAIL_ROWS)],
                    bounce.at[pl.ds(0, TAIL_ROWS)])
    pltpu.sync_copy(bounce.at[pl.ds(0, TAIL_ROWS)],
                    dst_ref.at[pl.ds(toff, TAIL_ROWS)])


def _sc_cnt_body(dst_e, zerosw, onesw, cnt0, cnt1, didx, onesv, cntb,
                 cntacc):
    c = lax.axis_index("c")
    s = lax.axis_index("s")
    wid = s * 2 + c
    base = s * TILE_ROWS
    start = wid * NCW

    pltpu.sync_copy(onesw, onesv)
    _staged_copy(zerosw, cntacc, cntb, base)
    plsc.subcore_barrier()

    def step(j, carry):
        off = pl.multiple_of(j * K, K)
        pltpu.sync_copy(dst_e.at[pl.ds(off, K)], didx)
        pltpu.sync_copy(onesv, cntacc.at[didx], add=True)
        return carry

    lax.fori_loop(start * K // K, (start + NCW), step, 0)
    plsc.subcore_barrier()

    @pl.when(c == 0)
    def _wb0():
        _staged_copy(cntacc, cnt0, cntb, base)

    @pl.when(c == 1)
    def _wb1():
        _staged_copy(cntacc, cnt1, cntb, base)


def _sc_adj_body(z, src_e, dst_e, out0, out1, sidx, didx, gbuf, rows,
                 gsem, acc):
    c = lax.axis_index("c")
    s = lax.axis_index("s")
    wid = s * 2 + c
    base = s * TILE_ROWS
    start = wid * NCW

    _staged_copy(z, acc, rows, base)
    plsc.subcore_barrier()

    def step(j, carry):
        off = pl.multiple_of(j * K, K)
        pltpu.sync_copy(src_e.at[pl.ds(off, K)], sidx)
        pltpu.sync_copy(dst_e.at[pl.ds(off, K)], didx)
        pltpu.async_copy(z.at[sidx], gbuf, gsem).wait()
        pltpu.sync_copy(gbuf, acc.at[didx], add=True)
        return carry

    lax.fori_loop(start, start + NCW, step, 0)
    plsc.subcore_barrier()

    @pl.when(c == 0)
    def _wb0():
        _staged_copy(acc, out0, rows, base)

    @pl.when(c == 1)
    def _wb1():
        _staged_copy(acc, out1, rows, base)


def _sc_mesh():
    return plsc.VectorSubcoreMesh(
        core_axis_name="c", subcore_axis_name="s", num_cores=2, num_subcores=16
    )


@functools.lru_cache(maxsize=None)
def _make_sc_cnt():
    return pl.kernel(
        _sc_cnt_body,
        out_type=[jax.ShapeDtypeStruct((NPAD, 128), jnp.float32)] * 2,
        mesh=_sc_mesh(),
        scratch_types=[
            pltpu.VMEM((K,), jnp.int32),
            pltpu.VMEM((K, 128), jnp.float32),
            pltpu.VMEM((K, 128), jnp.float32),
            pltpu.VMEM_SHARED((NPAD, 128), jnp.float32),
        ],
    )


@functools.lru_cache(maxsize=None)
def _make_sc_adj():
    return pl.kernel(
        _sc_adj_body,
        out_type=[jax.ShapeDtypeStruct((NPAD, 128), jnp.float32)] * 2,
        mesh=_sc_mesh(),
        scratch_types=[
            pltpu.VMEM((K,), jnp.int32),
            pltpu.VMEM((K,), jnp.int32),
            pltpu.VMEM((K, 128), jnp.float32),
            pltpu.VMEM((K, 128), jnp.float32),
            pltpu.SemaphoreType.DMA,
            pltpu.VMEM_SHARED((NPAD, 128), jnp.float32),
        ],
    )


def _lin_body(x_ref, w_ref, b_ref, o_ref):
    o_ref[...] = (
        jnp.dot(x_ref[...], w_ref[...], preferred_element_type=jnp.float32)
        + b_ref[...]
    )


def _tc_linear(x, W, b, bm=BM):
    m, d = x.shape
    h = W.shape[1]
    return pl.pallas_call(
        _lin_body,
        grid=(m // bm,),
        in_specs=[
            pl.BlockSpec((bm, d), lambda i: (i, 0)),
            pl.BlockSpec((d, h), lambda i: (0, 0)),
            pl.BlockSpec((1, h), lambda i: (0, 0)),
        ],
        out_specs=pl.BlockSpec((bm, h), lambda i: (i, 0)),
        out_shape=jax.ShapeDtypeStruct((m, h), jnp.float32),
    )(x, W, b.reshape(1, -1))


def _merge_rows(p0_ref, p1_ref, z_ref, c0_ref, c1_ref):
    u = jnp.maximum(p0_ref[...] + p1_ref[...] - z_ref[...], 0.0)
    inv = 1.0 / (1.0 + c0_ref[...][:, 0:1] + c1_ref[...][:, 0:1])
    return u * inv


def _mid_body(p0_ref, p1_ref, z_ref, c0_ref, c1_ref, w_ref, b_ref, o_ref):
    hrows = _merge_rows(p0_ref, p1_ref, z_ref, c0_ref, c1_ref)
    o_ref[...] = (
        jnp.dot(hrows, w_ref[...], preferred_element_type=jnp.float32)
        + b_ref[...]
    )


def _out_body(p0_ref, p1_ref, z_ref, c0_ref, c1_ref, w_ref, b_ref, o_ref):
    hrows = _merge_rows(p0_ref, p1_ref, z_ref, c0_ref, c1_ref)
    t = (
        jnp.dot(hrows, w_ref[...], preferred_element_type=jnp.float32)
        + b_ref[...]
    )
    mx = jnp.max(t, axis=1, keepdims=True)
    lse = jnp.log(jnp.sum(jnp.exp(t - mx), axis=1, keepdims=True)) + mx
    o_ref[...] = t - lse


def _tc_merge_mm(body, p0, p1, z, c0, c1, W, b, bm=BM):
    m, d = z.shape
    h = W.shape[1]
    return pl.pallas_call(
        body,
        grid=(m // bm,),
        in_specs=[
            pl.BlockSpec((bm, d), lambda i: (i, 0)),
            pl.BlockSpec((bm, d), lambda i: (i, 0)),
            pl.BlockSpec((bm, d), lambda i: (i, 0)),
            pl.BlockSpec((bm, d), lambda i: (i, 0)),
            pl.BlockSpec((bm, d), lambda i: (i, 0)),
            pl.BlockSpec((d, h), lambda i: (0, 0)),
            pl.BlockSpec((1, h), lambda i: (0, 0)),
        ],
        out_specs=pl.BlockSpec((bm, h), lambda i: (i, 0)),
        out_shape=jax.ShapeDtypeStruct((m, h), jnp.float32),
    )(p0, p1, z, c0, c1, W, b.reshape(1, -1))


def kernel(x, edge_index, W1, b1, W2, b2, W3, b3):
    sc_cnt = _make_sc_cnt()
    sc_adj = _make_sc_adj()

    dst_p = jnp.concatenate(
        [edge_index[0], jnp.full((EPAD,), NPAD - 1, jnp.int32)])
    src_p = jnp.concatenate([edge_index[1], jnp.zeros((EPAD,), jnp.int32)])

    zerosw = jnp.zeros((NPAD, 128), jnp.float32)
    onesw = jnp.ones((K, 128), jnp.float32)

    xp = jnp.pad(x, ((0, NPAD - N), (0, 0)))
    c0, c1 = sc_cnt(dst_p, zerosw, onesw)
    z1 = _tc_linear(xp, W1, b1)
    p0, p1 = sc_adj(z1, src_p, dst_p)
    z2 = _tc_merge_mm(_mid_body, p0, p1, z1, c0, c1, W2, b2)
    q0, q1 = sc_adj(z2, src_p, dst_p)
    return _tc_merge_mm(_out_body, q0, q1, z2, c0, c1, W3, b3)[:N]


# speedup vs baseline: 1.8846x; 1.8846x over previous
import functools

import jax
import jax.numpy as jnp
from jax import lax
from jax.experimental import pallas as pl
from jax.experimental.pallas import tpu as pltpu
from jax.experimental.pallas import tpu_sc as plsc

N = 10000
NPAD = 10112
E = 320000
K = 128
NCHUNKP = 2560
EPAD = NCHUNKP * K - E
NCW = NCHUNKP // 32
NSUB = 16
TILE_ROWS = NPAD // NSUB
NROWSTEP = TILE_ROWS // K
TAIL_ROWS = TILE_ROWS - NROWSTEP * K
BM = 1264


def _staged_copy(src_ref, dst_ref, bounce, base):
    def step(j, carry):
        off = pl.multiple_of(base + j * K, 8)
        pltpu.sync_copy(src_ref.at[pl.ds(off, K)], bounce)
        pltpu.sync_copy(bounce, dst_ref.at[pl.ds(off, K)])
        return carry
    lax.fori_loop(0, NROWSTEP, step, 0)
    toff = pl.multiple_of(base + NROWSTEP * K, 8)
    pltpu.sync_copy(src_ref.at[pl.ds(toff, TAIL_ROWS)],
                    bounce.at[pl.ds(0, TAIL_ROWS)])
    pltpu.sync_copy(bounce.at[pl.ds(0, TAIL_ROWS)],
                    dst_ref.at[pl.ds(toff, TAIL_ROWS)])


def _sc_cnt_body(dst_e, zerosw, onesw, cnt0, cnt1, didx, onesv, cntb,
                 cntacc):
    c = lax.axis_index("c")
    s = lax.axis_index("s")
    wid = s * 2 + c
    base = s * TILE_ROWS
    start = wid * NCW

    pltpu.sync_copy(onesw, onesv)
    _staged_copy(zerosw, cntacc, cntb, base)
    plsc.subcore_barrier()

    def step(j, carry):
        off = pl.multiple_of(j * K, K)
        pltpu.sync_copy(dst_e.at[pl.ds(off, K)], didx)
        pltpu.sync_copy(onesv, cntacc.at[didx], add=True)
        return carry

    lax.fori_loop(start * K // K, (start + NCW), step, 0)
    plsc.subcore_barrier()

    @pl.when(c == 0)
    def _wb0():
        _staged_copy(cntacc, cnt0, cntb, base)

    @pl.when(c == 1)
    def _wb1():
        _staged_copy(cntacc, cnt1, cntb, base)


def _sc_adj_body(z, src_e, dst_e, out0, out1, sidx, didx, gbuf, rows,
                 gsem, acc):
    c = lax.axis_index("c")
    s = lax.axis_index("s")
    wid = s * 2 + c
    base = s * TILE_ROWS
    start = wid * NCW

    _staged_copy(z, acc, rows, base)
    plsc.subcore_barrier()

    def step(j, carry):
        off = pl.multiple_of(j * K, K)
        pltpu.sync_copy(src_e.at[pl.ds(off, K)], sidx)
        pltpu.sync_copy(dst_e.at[pl.ds(off, K)], didx)
        pltpu.async_copy(z.at[sidx], gbuf, gsem).wait()
        pltpu.sync_copy(gbuf, acc.at[didx], add=True)
        return carry

    lax.fori_loop(start, start + NCW, step, 0)
    plsc.subcore_barrier()

    @pl.when(c == 0)
    def _wb0():
        _staged_copy(acc, out0, rows, base)

    @pl.when(c == 1)
    def _wb1():
        _staged_copy(acc, out1, rows, base)


def _sc_mesh():
    return plsc.VectorSubcoreMesh(
        core_axis_name="c", subcore_axis_name="s", num_cores=2, num_subcores=16
    )


@functools.lru_cache(maxsize=None)
def _make_sc_cnt():
    return pl.kernel(
        _sc_cnt_body,
        out_type=[jax.ShapeDtypeStruct((NPAD, 128), jnp.float32)] * 2,
        mesh=_sc_mesh(),
        scratch_types=[
            pltpu.VMEM((K,), jnp.int32),
            pltpu.VMEM((K, 128), jnp.float32),
            pltpu.VMEM((K, 128), jnp.float32),
            pltpu.VMEM_SHARED((NPAD, 128), jnp.float32),
        ],
    )


@functools.lru_cache(maxsize=None)
def _make_sc_adj():
    return pl.kernel(
        _sc_adj_body,
        out_type=[jax.ShapeDtypeStruct((NPAD, 128), jnp.float32)] * 2,
        mesh=_sc_mesh(),
        scratch_types=[
            pltpu.VMEM((K,), jnp.int32),
            pltpu.VMEM((K,), jnp.int32),
            pltpu.VMEM((K, 128), jnp.float32),
            pltpu.VMEM((K, 128), jnp.float32),
            pltpu.SemaphoreType.DMA,
            pltpu.VMEM_SHARED((NPAD, 128), jnp.float32),
        ],
    )


def _lin_body(x_ref, w_ref, b_ref, o_ref):
    o_ref[...] = (
        jnp.dot(x_ref[...], w_ref[...], preferred_element_type=jnp.float32)
        + b_ref[...]
    )


def _tc_linear(x, W, b, bm=BM):
    m, d = x.shape
    h = W.shape[1]
    return pl.pallas_call(
        _lin_body,
        grid=(m // bm,),
        in_specs=[
            pl.BlockSpec((bm, d), lambda i: (i, 0)),
            pl.BlockSpec((d, h), lambda i: (0, 0)),
            pl.BlockSpec((1, h), lambda i: (0, 0)),
        ],
        out_specs=pl.BlockSpec((bm, h), lambda i: (i, 0)),
        out_shape=jax.ShapeDtypeStruct((m, h), jnp.float32),
    )(x, W, b.reshape(1, -1))


def _merge_rows(p0_ref, p1_ref, z_ref, c0_ref, c1_ref):
    u = jnp.maximum(p0_ref[...] + p1_ref[...] - z_ref[...], 0.0)
    inv = 1.0 / (1.0 + c0_ref[...][:, 0:1] + c1_ref[...][:, 0:1])
    return u * inv


def _mid_body(p0_ref, p1_ref, z_ref, c0_ref, c1_ref, w_ref, b_ref, o_ref):
    hrows = _merge_rows(p0_ref, p1_ref, z_ref, c0_ref, c1_ref)
    o_ref[...] = (
        jnp.dot(hrows, w_ref[...], preferred_element_type=jnp.float32)
        + b_ref[...]
    )


def _out_body(p0_ref, p1_ref, z_ref, c0_ref, c1_ref, w_ref, b_ref, o_ref):
    hrows = _merge_rows(p0_ref, p1_ref, z_ref, c0_ref, c1_ref)
    t = (
        jnp.dot(hrows, w_ref[...], preferred_element_type=jnp.float32)
        + b_ref[...]
    )
    mx = jnp.max(t, axis=1, keepdims=True)
    lse = jnp.log(jnp.sum(jnp.exp(t - mx), axis=1, keepdims=True)) + mx
    o_ref[...] = t - lse


def _tc_merge_mm(body, p0, p1, z, c0, c1, W, b, bm=BM):
    m, d = z.shape
    h = W.shape[1]
    return pl.pallas_call(
        body,
        grid=(m // bm,),
        in_specs=[
            pl.BlockSpec((bm, d), lambda i: (i, 0)),
            pl.BlockSpec((bm, d), lambda i: (i, 0)),
            pl.BlockSpec((bm, d), lambda i: (i, 0)),
            pl.BlockSpec((bm, d), lambda i: (i, 0)),
            pl.BlockSpec((bm, d), lambda i: (i, 0)),
            pl.BlockSpec((d, h), lambda i: (0, 0)),
            pl.BlockSpec((1, h), lambda i: (0, 0)),
        ],
        out_specs=pl.BlockSpec((bm, h), lambda i: (i, 0)),
        out_shape=jax.ShapeDtypeStruct((m, h), jnp.float32),
    )(p0, p1, z, c0, c1, W, b.reshape(1, -1))


def kernel(x, edge_index, W1, b1, W2, b2, W3, b3):
    sc_cnt = _make_sc_cnt()
    sc_adj = _make_sc_adj()

    fake = jnp.arange(EPAD, dtype=jnp.int32)
    dst_p = jnp.concatenate([edge_index[0], N + fake % (NPAD - N)])
    src_p = jnp.concatenate([edge_index[1], fake % N])

    zerosw = jnp.zeros((NPAD, 128), jnp.float32)
    onesw = jnp.ones((K, 128), jnp.float32)

    xp = jnp.pad(x, ((0, NPAD - N), (0, 0)))
    c0, c1 = sc_cnt(dst_p, zerosw, onesw)
    z1 = _tc_linear(xp, W1, b1)
    p0, p1 = sc_adj(z1, src_p, dst_p)
    z2 = _tc_merge_mm(_mid_body, p0, p1, z1, c0, c1, W2, b2)
    q0, q1 = sc_adj(z2, src_p, dst_p)
    return _tc_merge_mm(_out_body, q0, q1, z2, c0, c1, W3, b3)[:N]


# speedup vs baseline: 2.8310x; 1.5022x over previous
import functools

import jax
import jax.numpy as jnp
from jax import lax
from jax.experimental import pallas as pl
from jax.experimental.pallas import tpu as pltpu
from jax.experimental.pallas import tpu_sc as plsc

N = 10000
NPAD = 10112
E = 320000
K = 128
NCHUNKP = 2560
EPAD = NCHUNKP * K - E
NCW = NCHUNKP // 32
NSUB = 16
TILE_ROWS = NPAD // NSUB
NROWSTEP = TILE_ROWS // K
TAIL_ROWS = TILE_ROWS - NROWSTEP * K
BM = 1264


def _staged_copy(src_ref, dst_ref, bounce, base):
    def step(j, carry):
        off = pl.multiple_of(base + j * K, 8)
        pltpu.sync_copy(src_ref.at[pl.ds(off, K)], bounce)
        pltpu.sync_copy(bounce, dst_ref.at[pl.ds(off, K)])
        return carry
    lax.fori_loop(0, NROWSTEP, step, 0)
    toff = pl.multiple_of(base + NROWSTEP * K, 8)
    pltpu.sync_copy(src_ref.at[pl.ds(toff, TAIL_ROWS)],
                    bounce.at[pl.ds(0, TAIL_ROWS)])
    pltpu.sync_copy(bounce.at[pl.ds(0, TAIL_ROWS)],
                    dst_ref.at[pl.ds(toff, TAIL_ROWS)])


def _sc_cnt_body(dst_e, zerosw, onesw, cnt0, cnt1, didx, onesv, cntb,
                 ssems, cntacc):
    c = lax.axis_index("c")
    s = lax.axis_index("s")
    wid = s * 2 + c
    base = s * TILE_ROWS
    start = wid * NCW

    pltpu.sync_copy(onesw, onesv)
    _staged_copy(zerosw, cntacc, cntb, base)
    plsc.subcore_barrier()

    def step(j, carry):
        pn = j % 2

        @pl.when(j >= 2)
        def _drain():
            pltpu.make_async_copy(onesw, onesv, ssems.at[pn]).wait()

        @pl.when(j <= NCW - 1)
        def _fire():
            off = pl.multiple_of((start + j) * K, K)
            pltpu.sync_copy(dst_e.at[pl.ds(off, K)], didx.at[pn])
            pltpu.async_copy(onesv, cntacc.at[didx.at[pn]], ssems.at[pn],
                             add=True)

        return carry

    lax.fori_loop(0, NCW + 2, step, 0)
    plsc.subcore_barrier()

    @pl.when(c == 0)
    def _wb0():
        _staged_copy(cntacc, cnt0, cntb, base)

    @pl.when(c == 1)
    def _wb1():
        _staged_copy(cntacc, cnt1, cntb, base)


def _sc_adj_body(z, src_e, dst_e, out0, out1, sidx, didx, gbuf, rows,
                 gsems, ssems, acc):
    c = lax.axis_index("c")
    s = lax.axis_index("s")
    wid = s * 2 + c
    base = s * TILE_ROWS
    start = wid * NCW

    _staged_copy(z, acc, rows, base)
    plsc.subcore_barrier()

    def step(j, carry):
        pn = j % 2
        pc = (j + 1) % 2

        @pl.when(j >= 2)
        def _drain_scatter():
            pltpu.make_async_copy(z.at[pl.ds(0, K)], gbuf.at[pn],
                                  ssems.at[pn]).wait()

        @pl.when(j <= NCW - 1)
        def _fire_gather():
            off = pl.multiple_of((start + j) * K, K)
            pltpu.sync_copy(src_e.at[pl.ds(off, K)], sidx.at[pn])
            pltpu.sync_copy(dst_e.at[pl.ds(off, K)], didx.at[pn])
            pltpu.async_copy(z.at[sidx.at[pn]], gbuf.at[pn], gsems.at[pn])

        @pl.when(j >= 1)
        def _fire_scatter():
            pltpu.make_async_copy(z.at[pl.ds(0, K)], gbuf.at[pc],
                                  gsems.at[pc]).wait()
            pltpu.async_copy(gbuf.at[pc], acc.at[didx.at[pc]],
                             ssems.at[pc], add=True)

        return carry

    lax.fori_loop(0, NCW + 1, step, 0)
    pltpu.make_async_copy(z.at[pl.ds(0, K)], gbuf.at[0], ssems.at[1]).wait()
    plsc.subcore_barrier()

    @pl.when(c == 0)
    def _wb0():
        _staged_copy(acc, out0, rows, base)

    @pl.when(c == 1)
    def _wb1():
        _staged_copy(acc, out1, rows, base)


def _sc_mesh():
    return plsc.VectorSubcoreMesh(
        core_axis_name="c", subcore_axis_name="s", num_cores=2, num_subcores=16
    )


@functools.lru_cache(maxsize=None)
def _make_sc_cnt():
    return pl.kernel(
        _sc_cnt_body,
        out_type=[jax.ShapeDtypeStruct((NPAD, 128), jnp.float32)] * 2,
        mesh=_sc_mesh(),
        scratch_types=[
            pltpu.VMEM((2, K), jnp.int32),
            pltpu.VMEM((K, 128), jnp.float32),
            pltpu.VMEM((K, 128), jnp.float32),
            pltpu.SemaphoreType.DMA((2,)),
            pltpu.VMEM_SHARED((NPAD, 128), jnp.float32),
        ],
    )


@functools.lru_cache(maxsize=None)
def _make_sc_adj():
    return pl.kernel(
        _sc_adj_body,
        out_type=[jax.ShapeDtypeStruct((NPAD, 128), jnp.float32)] * 2,
        mesh=_sc_mesh(),
        scratch_types=[
            pltpu.VMEM((2, K), jnp.int32),
            pltpu.VMEM((2, K), jnp.int32),
            pltpu.VMEM((2, K, 128), jnp.float32),
            pltpu.VMEM((K, 128), jnp.float32),
            pltpu.SemaphoreType.DMA((2,)),
            pltpu.SemaphoreType.DMA((2,)),
            pltpu.VMEM_SHARED((NPAD, 128), jnp.float32),
        ],
    )


def _lin_body(x_ref, w_ref, b_ref, o_ref):
    o_ref[...] = (
        jnp.dot(x_ref[...], w_ref[...], preferred_element_type=jnp.float32)
        + b_ref[...]
    )


def _tc_linear(x, W, b, bm=BM):
    m, d = x.shape
    h = W.shape[1]
    return pl.pallas_call(
        _lin_body,
        grid=(m // bm,),
        in_specs=[
            pl.BlockSpec((bm, d), lambda i: (i, 0)),
            pl.BlockSpec((d, h), lambda i: (0, 0)),
            pl.BlockSpec((1, h), lambda i: (0, 0)),
        ],
        out_specs=pl.BlockSpec((bm, h), lambda i: (i, 0)),
        out_shape=jax.ShapeDtypeStruct((m, h), jnp.float32),
    )(x, W, b.reshape(1, -1))


def _merge_rows(p0_ref, p1_ref, z_ref, c0_ref, c1_ref):
    u = jnp.maximum(p0_ref[...] + p1_ref[...] - z_ref[...], 0.0)
    inv = 1.0 / (1.0 + c0_ref[...][:, 0:1] + c1_ref[...][:, 0:1])
    return u * inv


def _mid_body(p0_ref, p1_ref, z_ref, c0_ref, c1_ref, w_ref, b_ref, o_ref):
    hrows = _merge_rows(p0_ref, p1_ref, z_ref, c0_ref, c1_ref)
    o_ref[...] = (
        jnp.dot(hrows, w_ref[...], preferred_element_type=jnp.float32)
        + b_ref[...]
    )


def _out_body(p0_ref, p1_ref, z_ref, c0_ref, c1_ref, w_ref, b_ref, o_ref):
    hrows = _merge_rows(p0_ref, p1_ref, z_ref, c0_ref, c1_ref)
    t = (
        jnp.dot(hrows, w_ref[...], preferred_element_type=jnp.float32)
        + b_ref[...]
    )
    mx = jnp.max(t, axis=1, keepdims=True)
    lse = jnp.log(jnp.sum(jnp.exp(t - mx), axis=1, keepdims=True)) + mx
    o_ref[...] = t - lse


def _tc_merge_mm(body, p0, p1, z, c0, c1, W, b, bm=BM):
    m, d = z.shape
    h = W.shape[1]
    return pl.pallas_call(
        body,
        grid=(m // bm,),
        in_specs=[
            pl.BlockSpec((bm, d), lambda i: (i, 0)),
            pl.BlockSpec((bm, d), lambda i: (i, 0)),
            pl.BlockSpec((bm, d), lambda i: (i, 0)),
            pl.BlockSpec((bm, d), lambda i: (i, 0)),
            pl.BlockSpec((bm, d), lambda i: (i, 0)),
            pl.BlockSpec((d, h), lambda i: (0, 0)),
            pl.BlockSpec((1, h), lambda i: (0, 0)),
        ],
        out_specs=pl.BlockSpec((bm, h), lambda i: (i, 0)),
        out_shape=jax.ShapeDtypeStruct((m, h), jnp.float32),
    )(p0, p1, z, c0, c1, W, b.reshape(1, -1))


def kernel(x, edge_index, W1, b1, W2, b2, W3, b3):
    sc_cnt = _make_sc_cnt()
    sc_adj = _make_sc_adj()

    fake = jnp.arange(EPAD, dtype=jnp.int32)
    dst_p = jnp.concatenate([edge_index[0], N + fake % (NPAD - N)])
    src_p = jnp.concatenate([edge_index[1], fake % N])

    zerosw = jnp.zeros((NPAD, 128), jnp.float32)
    onesw = jnp.ones((K, 128), jnp.float32)

    xp = jnp.pad(x, ((0, NPAD - N), (0, 0)))
    c0, c1 = sc_cnt(dst_p, zerosw, onesw)
    z1 = _tc_linear(xp, W1, b1)
    p0, p1 = sc_adj(z1, src_p, dst_p)
    z2 = _tc_merge_mm(_mid_body, p0, p1, z1, c0, c1, W2, b2)
    q0, q1 = sc_adj(z2, src_p, dst_p)
    return _tc_merge_mm(_out_body, q0, q1, z2, c0, c1, W3, b3)[:N]
